# baseline (speedup 1.0000x reference)
import jax
import jax.numpy as jnp
from jax import lax
from jax.experimental import pallas as pl
from jax.experimental.pallas import tpu as pltpu
from jax.experimental.pallas import tpu_sc as plsc

NC = 2
NS = 16
NW = NC * NS
CH = 128
NBUF = 8


def _body(idx_hbm, table_hbm, out_hbm, idx_v, rows_v, gsem, osem):
    cpw = idx_v.shape[0]
    wid = lax.axis_index("s") * NC + lax.axis_index("c")
    row_base = wid * cpw * CH

    pltpu.sync_copy(idx_hbm.at[wid], idx_v)

    def g_desc(chunk, slot):
        return pltpu.make_async_copy(
            table_hbm.at[idx_v.at[chunk]], rows_v.at[slot], gsem.at[slot])

    def s_desc(chunk, slot):
        return pltpu.make_async_copy(
            rows_v.at[slot], out_hbm.at[pl.ds(row_base + chunk * CH, CH)],
            osem.at[slot])

    for b in range(NBUF):
        g_desc(b, b).start()

    def outer(it, carry):
        jo = it * NBUF
        for b in range(NBUF):
            j = jo + b
            g_desc(j, b).wait()
            s_desc(j, b).start()
            jn = j + NBUF

            @pl.when(jn < cpw)
            def _():
                s_desc(j, b).wait()
                g_desc(jn, b).start()
        return carry

    lax.fori_loop(0, cpw // NBUF, outer, 0)

    for b in range(NBUF):
        s_desc(cpw - NBUF + b, b).wait()


def kernel(input_batch, table):
    batch, hist = input_batch.shape
    _, dim = table.shape
    total = batch * hist
    cpw = total // (NW * CH)
    idx = input_batch.reshape(NW, cpw, CH)
    mesh = plsc.VectorSubcoreMesh(
        core_axis_name="c", subcore_axis_name="s",
        num_cores=NC, num_subcores=NS)
    out = pl.kernel(
        _body,
        out_type=jax.ShapeDtypeStruct((total, dim), jnp.float32),
        mesh=mesh,
        compiler_params=pltpu.CompilerParams(use_tc_tiling_on_sc=False),
        scratch_types=[
            pltpu.VMEM((cpw, CH), jnp.int32),
            pltpu.VMEM((NBUF, CH, dim), jnp.float32),
            pltpu.SemaphoreType.DMA((NBUF,)),
            pltpu.SemaphoreType.DMA((NBUF,)),
        ],
    )(idx, table)
    return out.reshape(batch, hist, dim)


# speedup vs baseline: 1.9783x; 1.9783x over previous
import jax
import jax.numpy as jnp
from jax import lax
from jax.experimental import pallas as pl
from jax.experimental.pallas import tpu as pltpu
from jax.experimental.pallas import tpu_sc as plsc

NC = 2
NS = 16
NW = NC * NS
BQ = 4096
HIST = 50
DIM = 64
BATCH = 16384
VOCAB = 100000

NQ = BATCH // BQ
TASKS_PER_D = HIST * NQ
NP = DIM // NW
TOTAL = NP * TASKS_PER_D


def _decode(t):
    p = t // TASKS_PER_D
    r = t % TASKS_PER_D
    h = r // NQ
    qb = r % NQ
    return p, h, qb


def _body(idx_hbm, table_hbm, out_hbm, row_v, idx_v, out_v, isem, osem):
    wid = lax.axis_index("s") * NC + lax.axis_index("c")

    def idx_desc(t, slot):
        _, h, qb = _decode(t)
        return pltpu.make_async_copy(
            idx_hbm.at[h, pl.ds(qb * BQ, BQ)], idx_v.at[slot], isem.at[slot])

    def out_desc(t, slot):
        p, h, qb = _decode(t)
        d = wid + NW * p
        return pltpu.make_async_copy(
            out_v.at[slot], out_hbm.at[h, d, pl.ds(qb * BQ, BQ)], osem.at[slot])

    pltpu.sync_copy(table_hbm.at[wid], row_v)
    idx_desc(0, 0).start()
    idx_desc(1, 1).start()

    def step(it, carry):
        for b in range(2):
            t = it * 2 + b

            @pl.when(t == TASKS_PER_D)
            def _():
                pltpu.sync_copy(table_hbm.at[wid + NW], row_v)

            idx_desc(t, b).wait()

            @pl.when(t >= 2)
            def _():
                out_desc(t - 2, b).wait()

            @plsc.parallel_loop(0, BQ // 16, 1, unroll=8)
            def _(i):
                iv = idx_v[b, pl.ds(i * 16, 16)]
                out_v[b, pl.ds(i * 16, 16)] = plsc.load_gather(row_v, [iv])

            out_desc(t, b).start()

            @pl.when(t + 2 < TOTAL)
            def _():
                idx_desc(t + 2, b).start()
        return carry

    lax.fori_loop(0, TOTAL // 2, step, 0)
    out_desc(TOTAL - 2, 0).wait()
    out_desc(TOTAL - 1, 1).wait()


def kernel(input_batch, table):
    idx_t = input_batch.T
    table_t = table.T
    mesh = plsc.VectorSubcoreMesh(
        core_axis_name="c", subcore_axis_name="s",
        num_cores=NC, num_subcores=NS)
    out_phys = pl.kernel(
        _body,
        out_type=jax.ShapeDtypeStruct((HIST, DIM, BATCH), jnp.float32),
        mesh=mesh,
        compiler_params=pltpu.CompilerParams(
            use_tc_tiling_on_sc=True, needs_layout_passes=False),
        scratch_types=[
            pltpu.VMEM((VOCAB,), jnp.float32),
            pltpu.VMEM((2, BQ), jnp.int32),
            pltpu.VMEM((2, BQ), jnp.float32),
            pltpu.SemaphoreType.DMA((2,)),
            pltpu.SemaphoreType.DMA((2,)),
        ],
    )(idx_t, table_t)
    return jnp.transpose(out_phys, (2, 0, 1))


# speedup vs baseline: 2.6102x; 1.3194x over previous
import jax
import jax.numpy as jnp
from jax import lax
from jax.experimental import pallas as pl
from jax.experimental.pallas import tpu as pltpu
from jax.experimental.pallas import tpu_sc as plsc

NC = 2
NS = 16
NW = NC * NS
BQ = 4096
HIST = 50
DIM = 64
BATCH = 16384
VOCAB = 100000

NQ = BATCH // BQ
NP = DIM // NW
WINDOWS = ((0, 13), (13, 13), (26, 12), (38, 12))
HWMAX = max(hw for _, hw in WINDOWS)


def _body(idx_hbm, table_hbm, out_hbm, idx_sp, row_v, idx_v, out_v, isem, osem):
    cid = lax.axis_index("c")
    sid = lax.axis_index("s")
    wid = sid * NC + cid

    def idx_desc(h_rel, qb, slot):
        off = pl.multiple_of(h_rel * BATCH + qb * BQ, BQ)
        return pltpu.make_async_copy(
            idx_sp.at[pl.ds(off, BQ)], idx_v.at[slot], isem.at[slot])

    def out_desc(h, d, qb, slot):
        return pltpu.make_async_copy(
            out_v.at[slot], out_hbm.at[h, d, pl.ds(qb * BQ, BQ)], osem.at[slot])

    for p in range(NP):
        d = wid + NW * p
        pltpu.sync_copy(table_hbm.at[d], row_v)

        for w, (h0, hw) in enumerate(WINDOWS):
            if not (p == 0 and w == 0):
                plsc.subcore_barrier()
            for r in range(2):
                hh = sid + NS * r
                @pl.when(hh < hw)
                def _():
                    off = pl.multiple_of(hh * BATCH, BATCH)
                    pltpu.sync_copy(idx_hbm.at[h0 + hh],
                                    idx_sp.at[pl.ds(off, BATCH)])
            plsc.subcore_barrier()

            idx_desc(0, 0, 0).start()
            idx_desc(0, 1, 1).start()

            def h_step(h_rel, carry, p=p, w=w, d=d, h0=h0, hw=hw):
                for qb in range(NQ):
                    slot = qb % 2

                    idx_desc(0, 0, slot).wait()

                    if p == 0 and w == 0 and qb < 2:
                        @pl.when(h_rel > 0)
                        def _():
                            out_desc(0, d, qb, slot).wait()
                    else:
                        out_desc(0, d, qb, slot).wait()

                    @plsc.parallel_loop(0, BQ // 16, 1, unroll=8)
                    def _(i):
                        iv = idx_v[slot, pl.ds(i * 16, 16)]
                        out_v[slot, pl.ds(i * 16, 16)] = plsc.load_gather(
                            row_v, [iv])

                    out_desc(h0 + h_rel, d, qb, slot).start()

                    if qb < 2:
                        idx_desc(h_rel, qb + 2, slot).start()
                    else:
                        @pl.when(h_rel < hw - 1)
                        def _():
                            idx_desc(h_rel + 1, qb - 2, slot).start()
                return carry

            lax.fori_loop(0, hw, h_step, 0)

    out_desc(HIST - 1, wid + NW, NQ - 2, 0).wait()
    out_desc(HIST - 1, wid + NW, NQ - 1, 1).wait()


def kernel(input_batch, table):
    idx_t = input_batch.T
    table_t = table.T
    mesh = plsc.VectorSubcoreMesh(
        core_axis_name="c", subcore_axis_name="s",
        num_cores=NC, num_subcores=NS)
    out_phys = pl.kernel(
        _body,
        out_type=jax.ShapeDtypeStruct((HIST, DIM, BATCH), jnp.float32),
        mesh=mesh,
        compiler_params=pltpu.CompilerParams(
            use_tc_tiling_on_sc=True, needs_layout_passes=False),
        scratch_types=[
            pltpu.VMEM_SHARED((HWMAX * BATCH,), jnp.int32),
            pltpu.VMEM((VOCAB,), jnp.float32),
            pltpu.VMEM((2, BQ), jnp.int32),
            pltpu.VMEM((2, BQ), jnp.float32),
            pltpu.SemaphoreType.DMA((2,)),
            pltpu.SemaphoreType.DMA((2,)),
        ],
    )(idx_t, table_t)
    return jnp.transpose(out_phys, (2, 0, 1))


# speedup vs baseline: 2.6210x; 1.0041x over previous
import jax
import jax.numpy as jnp
from jax import lax
from jax.experimental import pallas as pl
from jax.experimental.pallas import tpu as pltpu
from jax.experimental.pallas import tpu_sc as plsc

NC = 2
NS = 16
NW = NC * NS
BQ = 4096
HIST = 50
DIM = 64
BATCH = 16384
VOCAB = 100000

NQ = BATCH // BQ
NP = DIM // NW
WINDOWS = ((0, 13), (13, 13), (26, 12), (38, 12))
HWMAX = max(hw for _, hw in WINDOWS)


def _body(idx_hbm, table_hbm, out_hbm, idx_sp, row_v, idx_v, out_v, isem, osem):
    cid = lax.axis_index("c")
    sid = lax.axis_index("s")
    wid = sid * NC + cid

    def idx_desc(h_rel, qb, slot):
        off = pl.multiple_of(h_rel * BATCH + qb * BQ, BQ)
        return pltpu.make_async_copy(
            idx_sp.at[pl.ds(off, BQ)], idx_v.at[slot], isem.at[slot])

    def out_desc(h, d, qb, slot):
        return pltpu.make_async_copy(
            out_v.at[slot], out_hbm.at[h, d, pl.ds(qb * BQ, BQ)], osem.at[slot])

    for p in range(NP):
        d = wid + NW * p
        pltpu.sync_copy(table_hbm.at[d], row_v)

        for w, (h0, hw) in enumerate(WINDOWS):
            if not (p == 0 and w == 0):
                plsc.subcore_barrier()
            for r in range(2):
                hh = sid + NS * r
                @pl.when(hh < hw)
                def _():
                    off = pl.multiple_of(hh * BATCH, BATCH)
                    pltpu.sync_copy(idx_hbm.at[h0 + hh],
                                    idx_sp.at[pl.ds(off, BATCH)])
            plsc.subcore_barrier()

            idx_desc(0, 0, 0).start()
            idx_desc(0, 1, 1).start()

            def h_step(h_rel, carry, p=p, w=w, d=d, h0=h0, hw=hw):
                for qb in range(NQ):
                    slot = qb % 2

                    idx_desc(0, 0, slot).wait()

                    if p == 0 and w == 0 and qb < 2:
                        @pl.when(h_rel > 0)
                        def _():
                            out_desc(0, d, qb, slot).wait()
                    else:
                        out_desc(0, d, qb, slot).wait()

                    @plsc.parallel_loop(0, BQ // 16, 1, unroll=16)
                    def _(i):
                        iv = idx_v[slot, pl.ds(i * 16, 16)]
                        out_v[slot, pl.ds(i * 16, 16)] = plsc.load_gather(
                            row_v, [iv])

                    out_desc(h0 + h_rel, d, qb, slot).start()

                    if qb < 2:
                        idx_desc(h_rel, qb + 2, slot).start()
                    else:
                        @pl.when(h_rel < hw - 1)
                        def _():
                            idx_desc(h_rel + 1, qb - 2, slot).start()
                return carry

            lax.fori_loop(0, hw, h_step, 0)

    out_desc(HIST - 1, wid + NW, NQ - 2, 0).wait()
    out_desc(HIST - 1, wid + NW, NQ - 1, 1).wait()


def kernel(input_batch, table):
    idx_t = input_batch.T
    table_t = table.T
    mesh = plsc.VectorSubcoreMesh(
        core_axis_name="c", subcore_axis_name="s",
        num_cores=NC, num_subcores=NS)
    out_phys = pl.kernel(
        _body,
        out_type=jax.ShapeDtypeStruct((HIST, DIM, BATCH), jnp.float32),
        mesh=mesh,
        compiler_params=pltpu.CompilerParams(
            use_tc_tiling_on_sc=True, needs_layout_passes=False),
        scratch_types=[
            pltpu.VMEM_SHARED((HWMAX * BATCH,), jnp.int32),
            pltpu.VMEM((VOCAB,), jnp.float32),
            pltpu.VMEM((2, BQ), jnp.int32),
            pltpu.VMEM((2, BQ), jnp.float32),
            pltpu.SemaphoreType.DMA((2,)),
            pltpu.SemaphoreType.DMA((2,)),
        ],
    )(idx_t, table_t)
    return jnp.transpose(out_phys, (2, 0, 1))


# speedup vs baseline: 2.6486x; 1.0105x over previous
import jax
import jax.numpy as jnp
from jax import lax
from jax.experimental import pallas as pl
from jax.experimental.pallas import tpu as pltpu
from jax.experimental.pallas import tpu_sc as plsc

NC = 2
NS = 16
NW = NC * NS
BQ = 4096
HIST = 50
DIM = 64
BATCH = 16384
VOCAB = 100000

NQ = BATCH // BQ
NP = DIM // NW
WINDOWS = ((0, 13), (13, 13), (26, 12), (38, 12))
HWMAX = max(hw for _, hw in WINDOWS)


def _body(idx_hbm, table_hbm, out_hbm,
          idx_sp, row_v, idx_v, out_v, isem, osem, rsem):
    cid = lax.axis_index("c")
    sid = lax.axis_index("s")
    wid = sid * NC + cid

    def idx_desc(h_rel, qb, slot):
        off = pl.multiple_of(h_rel * BATCH + qb * BQ, BQ)
        return pltpu.make_async_copy(
            idx_sp.at[pl.ds(off, BQ)], idx_v.at[slot], isem.at[slot])

    def out_desc(h, d, qb, slot):
        return pltpu.make_async_copy(
            out_v.at[slot], out_hbm.at[h, d, pl.ds(qb * BQ, BQ)], osem.at[slot])

    for p in range(NP):
        d = wid + NW * p
        pltpu.make_async_copy(table_hbm.at[d], row_v, rsem).start()

        for w, (h0, hw) in enumerate(WINDOWS):
            if not (p == 0 and w == 0):
                plsc.subcore_barrier()
            for r in range(2):
                hh = sid + NS * r
                @pl.when(hh < hw)
                def _():
                    off = pl.multiple_of(hh * BATCH, BATCH)
                    pltpu.sync_copy(idx_hbm.at[h0 + hh],
                                    idx_sp.at[pl.ds(off, BATCH)])
            plsc.subcore_barrier()

            idx_desc(0, 0, 0).start()
            idx_desc(0, 1, 1).start()
            if w == 0:
                pltpu.make_async_copy(table_hbm.at[d], row_v, rsem).wait()

            def h_step(h_rel, carry, p=p, w=w, d=d, h0=h0, hw=hw):
                for qb in range(NQ):
                    slot = qb % 2

                    idx_desc(0, 0, slot).wait()

                    if p == 0 and w == 0 and qb < 2:
                        @pl.when(h_rel > 0)
                        def _():
                            out_desc(0, d, qb, slot).wait()
                    else:
                        out_desc(0, d, qb, slot).wait()

                    @plsc.parallel_loop(0, BQ // 16, 1, unroll=16)
                    def _(i):
                        iv = idx_v[slot, pl.ds(i * 16, 16)]
                        out_v[slot, pl.ds(i * 16, 16)] = plsc.load_gather(
                            row_v, [iv])

                    out_desc(h0 + h_rel, d, qb, slot).start()

                    if qb < 2:
                        idx_desc(h_rel, qb + 2, slot).start()
                    else:
                        @pl.when(h_rel < hw - 1)
                        def _():
                            idx_desc(h_rel + 1, qb - 2, slot).start()
                return carry

            lax.fori_loop(0, hw, h_step, 0)

    out_desc(HIST - 1, wid + NW, NQ - 2, 0).wait()
    out_desc(HIST - 1, wid + NW, NQ - 1, 1).wait()


def kernel(input_batch, table):
    idx_t = input_batch.T
    table_t = table.T
    mesh = plsc.VectorSubcoreMesh(
        core_axis_name="c", subcore_axis_name="s",
        num_cores=NC, num_subcores=NS)
    out_phys = pl.kernel(
        _body,
        out_type=jax.ShapeDtypeStruct((HIST, DIM, BATCH), jnp.float32),
        mesh=mesh,
        compiler_params=pltpu.CompilerParams(
            use_tc_tiling_on_sc=True, needs_layout_passes=False),
        scratch_types=[
            pltpu.VMEM_SHARED((HWMAX * BATCH,), jnp.int32),
            pltpu.VMEM((VOCAB,), jnp.float32),
            pltpu.VMEM((2, BQ), jnp.int32),
            pltpu.VMEM((2, BQ), jnp.float32),
            pltpu.SemaphoreType.DMA((2,)),
            pltpu.SemaphoreType.DMA((2,)),
            pltpu.SemaphoreType.DMA,
        ],
    )(idx_t, table_t)
    return jnp.transpose(out_phys, (2, 0, 1))
